# baseline (speedup 1.0000x reference)
import jax
import jax.numpy as jnp
from jax import lax
from jax.experimental import pallas as pl
from jax.experimental.pallas import tpu as pltpu
from jax.experimental.pallas import tpu_sc as plsc

N = 10000
E = 320000
D = 128
ALPHA = 0.2

NC = 2
NS = 16
NW = NC * NS
CH = 80
GS = 5
NG = E // (NW * GS * CH)
NB = 3
ROWS_PER_TILE = N // NS


def _pre_body(x_ref, w_ref, al_ref, ar_ref, h_ref, hl_ref, hr_ref, h16_ref):
    h = jnp.dot(x_ref[...], w_ref[...], preferred_element_type=jnp.float32)
    h = jnp.where(jnp.isnan(h), 0.0, h)
    h_ref[...] = h
    hl = jnp.sum(h * al_ref[...][None, :], axis=1)
    hr = jnp.sum(h * ar_ref[...][None, :], axis=1)
    hl_ref[...] = hl
    hr_ref[...] = hr
    h16_ref[...] = jnp.full((16,), jnp.max(hr), jnp.float32)


_pre_call = pl.pallas_call(
    _pre_body,
    out_shape=[
        jax.ShapeDtypeStruct((N, D), jnp.float32),
        jax.ShapeDtypeStruct((N,), jnp.float32),
        jax.ShapeDtypeStruct((N,), jnp.float32),
        jax.ShapeDtypeStruct((16,), jnp.float32),
    ],
)


def _sc_body(h_hbm, row_hbm, col_hbm, hl_hbm, hr_hbm, h16_hbm,
             acc_hbm, den_hbm,
             h16_v, rowi_v, coli_v, w_v, zidx_v, hl3_v, hr3_v, rows3_v,
             acc_sh, den_sh,
             gsem0, gsem1, gsem2, ssem0, ssem1, ssem2,
             lsem0, lsem1, lsem2, rsem0, rsem1, rsem2,
             isem, dsem0, dsem1):
    gsem = (gsem0, gsem1, gsem2)
    ssem = (ssem0, ssem1, ssem2)
    lsem = (lsem0, lsem1, lsem2)
    rsem = (rsem0, rsem1, rsem2)
    dsem = (dsem0, dsem1)
    c = lax.axis_index("c")
    s = lax.axis_index("s")
    wid = s * NC + c

    pltpu.sync_copy(h16_hbm, h16_v)

    def _zero_rows(k, carry):
        for b in range(NB):
            for v in range(D // 16):
                rows3_v[b, k, pl.ds(v * 16, 16)] = jnp.zeros((16,), jnp.float32)
        return carry
    lax.fori_loop(0, CH, _zero_rows, 0)
    for k in range(CH // 16):
        zidx_v[pl.ds(k * 16, 16)] = jnp.zeros((16,), jnp.int32)
        for jp in range(2):
            w_v[jp, pl.ds(k * 16, 16)] = jnp.zeros((16,), jnp.float32)

    lo = s * ROWS_PER_TILE
    for t in range(ROWS_PER_TILE // CH):
        pltpu.sync_copy(rows3_v.at[0], acc_sh.at[pl.ds(lo + t * CH, CH)])
    rem = ROWS_PER_TILE % CH
    if rem:
        pltpu.sync_copy(rows3_v.at[0, pl.ds(0, rem)],
                        acc_sh.at[pl.ds(lo + (ROWS_PER_TILE // CH) * CH, rem)])
    @pl.when(s == 0)
    def _zero_den():
        def _zd(k, carry):
            pltpu.sync_copy(rows3_v.at[0, 0], den_sh.at[pl.ds(k * D, D)])
            return carry
        lax.fori_loop(0, N // D, _zd, 0)
        pltpu.sync_copy(rows3_v.at[0, 0, pl.ds(0, N % D)],
                        den_sh.at[pl.ds((N // D) * D, N % D)])
    plsc.subcore_barrier()

    for b in range(NB):
        pltpu.async_copy(rows3_v.at[b], acc_sh.at[zidx_v], ssem[b], add=True)
    for jp in range(2):
        pltpu.async_copy(w_v.at[jp], den_sh.at[zidx_v], dsem[jp], add=True)
    pltpu.async_copy(row_hbm.at[wid, 0], rowi_v.at[0], isem)
    pltpu.async_copy(col_hbm.at[wid, 0], coli_v.at[0], isem)

    hsplat = h16_v[pl.ds(0, 16)]

    def _issue(pg, b, j):
        pltpu.make_async_copy(rows3_v.at[b], acc_sh.at[zidx_v], ssem[b]).wait()
        pltpu.async_copy(hl_hbm.at[rowi_v.at[pg, j]], hl3_v.at[b], lsem[b])
        pltpu.async_copy(hr_hbm.at[coli_v.at[pg, j]], hr3_v.at[b], rsem[b])
        pltpu.async_copy(h_hbm.at[coli_v.at[pg, j]], rows3_v.at[b], gsem[b])

    def _group(g, carry):
        pg = g % 2
        pltpu.make_async_copy(row_hbm.at[wid, g], rowi_v.at[pg], isem).wait()
        pltpu.make_async_copy(col_hbm.at[wid, g], coli_v.at[pg], isem).wait()
        for j in range(min(NB, GS)):
            _issue(pg, j % NB, j)
        for j in range(GS):
            b = j % NB
            jp = j % 2
            pltpu.make_async_copy(hl_hbm.at[rowi_v.at[pg, j]], hl3_v.at[b],
                                  lsem[b]).wait()
            pltpu.make_async_copy(hr_hbm.at[coli_v.at[pg, j]], hr3_v.at[b],
                                  rsem[b]).wait()
            pltpu.make_async_copy(w_v.at[jp], den_sh.at[zidx_v],
                                  dsem[jp]).wait()
            for k in range(CH // 16):
                sl = pl.ds(k * 16, 16)
                a = hl3_v[b, sl]
                t0 = a + hr3_v[b, sl]
                e = jnp.where(t0 >= 0.0, t0, t0 * ALPHA)
                tm = a + hsplat
                m = jnp.where(tm >= 0.0, tm, tm * ALPHA)
                w_v[jp, sl] = jnp.exp(e - m)
            pltpu.async_copy(w_v.at[jp], den_sh.at[rowi_v.at[pg, j]],
                             dsem[jp], add=True)
            if j == 1:
                @pl.when(g + 1 < NG)
                def _prefetch():
                    pltpu.async_copy(row_hbm.at[wid, g + 1],
                                     rowi_v.at[1 - pg], isem)
                    pltpu.async_copy(col_hbm.at[wid, g + 1],
                                     coli_v.at[1 - pg], isem)
            pltpu.make_async_copy(h_hbm.at[coli_v.at[pg, j]], rows3_v.at[b],
                                  gsem[b]).wait()

            @plsc.parallel_loop(0, CH, 1, unroll=16)
            def _scale(jj):
                ws = plsc.load_gather(w_v.at[jp], [jnp.full((16,), jj,
                                                            jnp.int32)])
                for v in range(D // 16):
                    sl = pl.ds(v * 16, 16)
                    rows3_v[b, jj, sl] = rows3_v[b, jj, sl] * ws

            pltpu.async_copy(rows3_v.at[b], acc_sh.at[rowi_v.at[pg, j]],
                             ssem[b], add=True)
            if j + NB < GS:
                _issue(pg, (j + NB) % NB, j + NB)
        return carry

    lax.fori_loop(0, NG, _group, 0)

    for b in range(NB):
        pltpu.make_async_copy(rows3_v.at[b], acc_sh.at[zidx_v], ssem[b]).wait()
    for jp in range(2):
        pltpu.make_async_copy(w_v.at[jp], den_sh.at[zidx_v], dsem[jp]).wait()
    plsc.subcore_barrier()
    pltpu.sync_copy(acc_sh.at[pl.ds(lo, ROWS_PER_TILE)], acc_hbm.at[c, s])
    @pl.when(s == 0)
    def _out_den():
        pltpu.sync_copy(den_sh, den_hbm.at[c])


_sc_call = pl.kernel(
    _sc_body,
    out_type=(
        jax.ShapeDtypeStruct((NC, NS, ROWS_PER_TILE, D), jnp.float32),
        jax.ShapeDtypeStruct((NC, N), jnp.float32),
    ),
    mesh=plsc.VectorSubcoreMesh(core_axis_name="c", subcore_axis_name="s"),
    compiler_params=pltpu.CompilerParams(needs_layout_passes=False),
    scratch_types=[
        pltpu.VMEM((16,), jnp.float32),
        pltpu.VMEM((2, GS, CH), jnp.int32),
        pltpu.VMEM((2, GS, CH), jnp.int32),
        pltpu.VMEM((2, CH), jnp.float32),
        pltpu.VMEM((CH,), jnp.int32),
        pltpu.VMEM((NB, CH), jnp.float32),
        pltpu.VMEM((NB, CH), jnp.float32),
        pltpu.VMEM((NB, CH, D), jnp.float32),
        pltpu.VMEM_SHARED((N, D), jnp.float32),
        pltpu.VMEM_SHARED((N,), jnp.float32),
        pltpu.SemaphoreType.DMA,
        pltpu.SemaphoreType.DMA,
        pltpu.SemaphoreType.DMA,
        pltpu.SemaphoreType.DMA,
        pltpu.SemaphoreType.DMA,
        pltpu.SemaphoreType.DMA,
        pltpu.SemaphoreType.DMA,
        pltpu.SemaphoreType.DMA,
        pltpu.SemaphoreType.DMA,
        pltpu.SemaphoreType.DMA,
        pltpu.SemaphoreType.DMA,
        pltpu.SemaphoreType.DMA,
        pltpu.SemaphoreType.DMA,
        pltpu.SemaphoreType.DMA,
        pltpu.SemaphoreType.DMA,
    ],
)


def _comb_body(acc_ref, den_ref, out_ref):
    a = acc_ref[0] + acc_ref[1]
    dsum = den_ref[0] + den_ref[1]
    out_ref[...] = a / (dsum[:, None] + 1e-16)


_comb_call = pl.pallas_call(
    _comb_body,
    out_shape=jax.ShapeDtypeStruct((N, D), jnp.float32),
)


@jax.jit
def kernel(x, edge_index, W, a_l, a_r):
    al = a_l.reshape(-1).astype(jnp.float32)
    ar = a_r.reshape(-1).astype(jnp.float32)
    h, hl, hr, h16 = _pre_call(x, W, al, ar)
    row4 = edge_index[0].reshape(NW, NG, GS, CH)
    col4 = edge_index[1].reshape(NW, NG, GS, CH)
    acc, den = _sc_call(h, row4, col4, hl, hr, h16)
    return _comb_call(acc.reshape(NC, N, D), den)


# speedup vs baseline: 1.0759x; 1.0759x over previous
import jax
import jax.numpy as jnp
from jax import lax
from jax.experimental import pallas as pl
from jax.experimental.pallas import tpu as pltpu
from jax.experimental.pallas import tpu_sc as plsc

N = 10000
E = 320000
D = 128
ALPHA = 0.2

NC = 2
NS = 16
NW = NC * NS
CH = 80
GS = 5
NG = E // (NW * GS * CH)
NB = 3
ROWS_PER_TILE = N // NS


def _pre_body(x_ref, w_ref, al_ref, ar_ref, h_ref, hl_ref, hr_ref, h16_ref):
    h = jnp.dot(x_ref[...], w_ref[...], preferred_element_type=jnp.float32)
    h = jnp.where(jnp.isnan(h), 0.0, h)
    h_ref[...] = h
    hl = jnp.sum(h * al_ref[...][None, :], axis=1)
    hr = jnp.sum(h * ar_ref[...][None, :], axis=1)
    hl_ref[...] = hl
    hr_ref[...] = hr
    h16_ref[...] = jnp.full((16,), jnp.max(hr), jnp.float32)


_pre_call = pl.pallas_call(
    _pre_body,
    out_shape=[
        jax.ShapeDtypeStruct((N, D), jnp.float32),
        jax.ShapeDtypeStruct((N,), jnp.float32),
        jax.ShapeDtypeStruct((N,), jnp.float32),
        jax.ShapeDtypeStruct((16,), jnp.float32),
    ],
)


def _sc_body(h_hbm, row_hbm, col_hbm, hl_hbm, hr_hbm, h16_hbm,
             acc_hbm, den_hbm,
             h16_v, rowi_v, coli_v, w_v, zidx_v, hl3_v, hr3_v, rows3_v,
             acc_sh, den_sh, hl_sh, hr_sh,
             gsem0, gsem1, gsem2, ssem0, ssem1, ssem2,
             lsem0, lsem1, lsem2, rsem0, rsem1, rsem2,
             isem, dsem0, dsem1):
    gsem = (gsem0, gsem1, gsem2)
    ssem = (ssem0, ssem1, ssem2)
    lsem = (lsem0, lsem1, lsem2)
    rsem = (rsem0, rsem1, rsem2)
    dsem = (dsem0, dsem1)
    c = lax.axis_index("c")
    s = lax.axis_index("s")
    wid = s * NC + c

    pltpu.sync_copy(h16_hbm, h16_v)
    @pl.when(s == 1)
    def _stage_hlr():
        pltpu.sync_copy(hl_hbm, hl_sh)
        pltpu.sync_copy(hr_hbm, hr_sh)

    def _zero_rows(k, carry):
        for b in range(NB):
            for v in range(D // 16):
                rows3_v[b, k, pl.ds(v * 16, 16)] = jnp.zeros((16,), jnp.float32)
        return carry
    lax.fori_loop(0, CH, _zero_rows, 0)
    for k in range(CH // 16):
        zidx_v[pl.ds(k * 16, 16)] = jnp.zeros((16,), jnp.int32)
        for jp in range(2):
            w_v[jp, pl.ds(k * 16, 16)] = jnp.zeros((16,), jnp.float32)

    lo = s * ROWS_PER_TILE
    for t in range(ROWS_PER_TILE // CH):
        pltpu.sync_copy(rows3_v.at[0], acc_sh.at[pl.ds(lo + t * CH, CH)])
    rem = ROWS_PER_TILE % CH
    if rem:
        pltpu.sync_copy(rows3_v.at[0, pl.ds(0, rem)],
                        acc_sh.at[pl.ds(lo + (ROWS_PER_TILE // CH) * CH, rem)])
    @pl.when(s == 0)
    def _zero_den():
        def _zd(k, carry):
            pltpu.sync_copy(rows3_v.at[0, 0], den_sh.at[pl.ds(k * D, D)])
            return carry
        lax.fori_loop(0, N // D, _zd, 0)
        pltpu.sync_copy(rows3_v.at[0, 0, pl.ds(0, N % D)],
                        den_sh.at[pl.ds((N // D) * D, N % D)])
    plsc.subcore_barrier()

    for b in range(NB):
        pltpu.async_copy(rows3_v.at[b], acc_sh.at[zidx_v], ssem[b], add=True)
    for jp in range(2):
        pltpu.async_copy(w_v.at[jp], den_sh.at[zidx_v], dsem[jp], add=True)
    pltpu.async_copy(row_hbm.at[wid, 0], rowi_v.at[0], isem)
    pltpu.async_copy(col_hbm.at[wid, 0], coli_v.at[0], isem)

    hsplat = h16_v[pl.ds(0, 16)]

    def _issue(pg, b, j):
        pltpu.make_async_copy(rows3_v.at[b], acc_sh.at[zidx_v], ssem[b]).wait()
        pltpu.async_copy(hl_sh.at[rowi_v.at[pg, j]], hl3_v.at[b], lsem[b])
        pltpu.async_copy(hr_sh.at[coli_v.at[pg, j]], hr3_v.at[b], rsem[b])
        pltpu.async_copy(h_hbm.at[coli_v.at[pg, j]], rows3_v.at[b], gsem[b])

    def _group(g, carry):
        pg = g % 2
        pltpu.make_async_copy(row_hbm.at[wid, g], rowi_v.at[pg], isem).wait()
        pltpu.make_async_copy(col_hbm.at[wid, g], coli_v.at[pg], isem).wait()
        for j in range(min(NB, GS)):
            _issue(pg, j % NB, j)
        for j in range(GS):
            b = j % NB
            jp = j % 2
            pltpu.make_async_copy(hl_sh.at[rowi_v.at[pg, j]], hl3_v.at[b],
                                  lsem[b]).wait()
            pltpu.make_async_copy(hr_sh.at[coli_v.at[pg, j]], hr3_v.at[b],
                                  rsem[b]).wait()
            pltpu.make_async_copy(w_v.at[jp], den_sh.at[zidx_v],
                                  dsem[jp]).wait()
            for k in range(CH // 16):
                sl = pl.ds(k * 16, 16)
                a = hl3_v[b, sl]
                t0 = a + hr3_v[b, sl]
                e = jnp.where(t0 >= 0.0, t0, t0 * ALPHA)
                tm = a + hsplat
                m = jnp.where(tm >= 0.0, tm, tm * ALPHA)
                w_v[jp, sl] = jnp.exp(e - m)
            pltpu.async_copy(w_v.at[jp], den_sh.at[rowi_v.at[pg, j]],
                             dsem[jp], add=True)
            if j == 1:
                @pl.when(g + 1 < NG)
                def _prefetch():
                    pltpu.async_copy(row_hbm.at[wid, g + 1],
                                     rowi_v.at[1 - pg], isem)
                    pltpu.async_copy(col_hbm.at[wid, g + 1],
                                     coli_v.at[1 - pg], isem)
            pltpu.make_async_copy(h_hbm.at[coli_v.at[pg, j]], rows3_v.at[b],
                                  gsem[b]).wait()

            @plsc.parallel_loop(0, CH, 1, unroll=8)
            def _scale(jj):
                ws = plsc.load_gather(w_v.at[jp], [jnp.full((16,), jj,
                                                            jnp.int32)])
                for v in range(D // 16):
                    sl = pl.ds(v * 16, 16)
                    rows3_v[b, jj, sl] = rows3_v[b, jj, sl] * ws

            pltpu.async_copy(rows3_v.at[b], acc_sh.at[rowi_v.at[pg, j]],
                             ssem[b], add=True)
            if j + NB < GS:
                _issue(pg, (j + NB) % NB, j + NB)
        return carry

    lax.fori_loop(0, NG, _group, 0)

    for b in range(NB):
        pltpu.make_async_copy(rows3_v.at[b], acc_sh.at[zidx_v], ssem[b]).wait()
    for jp in range(2):
        pltpu.make_async_copy(w_v.at[jp], den_sh.at[zidx_v], dsem[jp]).wait()
    plsc.subcore_barrier()
    pltpu.sync_copy(acc_sh.at[pl.ds(lo, ROWS_PER_TILE)], acc_hbm.at[c, s])
    @pl.when(s == 0)
    def _out_den():
        pltpu.sync_copy(den_sh, den_hbm.at[c])


_sc_call = pl.kernel(
    _sc_body,
    out_type=(
        jax.ShapeDtypeStruct((NC, NS, ROWS_PER_TILE, D), jnp.float32),
        jax.ShapeDtypeStruct((NC, N), jnp.float32),
    ),
    mesh=plsc.VectorSubcoreMesh(core_axis_name="c", subcore_axis_name="s"),
    compiler_params=pltpu.CompilerParams(needs_layout_passes=False),
    scratch_types=[
        pltpu.VMEM((16,), jnp.float32),
        pltpu.VMEM((2, GS, CH), jnp.int32),
        pltpu.VMEM((2, GS, CH), jnp.int32),
        pltpu.VMEM((2, CH), jnp.float32),
        pltpu.VMEM((CH,), jnp.int32),
        pltpu.VMEM((NB, CH), jnp.float32),
        pltpu.VMEM((NB, CH), jnp.float32),
        pltpu.VMEM((NB, CH, D), jnp.float32),
        pltpu.VMEM_SHARED((N, D), jnp.float32),
        pltpu.VMEM_SHARED((N,), jnp.float32),
        pltpu.VMEM_SHARED((N,), jnp.float32),
        pltpu.VMEM_SHARED((N,), jnp.float32),
        pltpu.SemaphoreType.DMA,
        pltpu.SemaphoreType.DMA,
        pltpu.SemaphoreType.DMA,
        pltpu.SemaphoreType.DMA,
        pltpu.SemaphoreType.DMA,
        pltpu.SemaphoreType.DMA,
        pltpu.SemaphoreType.DMA,
        pltpu.SemaphoreType.DMA,
        pltpu.SemaphoreType.DMA,
        pltpu.SemaphoreType.DMA,
        pltpu.SemaphoreType.DMA,
        pltpu.SemaphoreType.DMA,
        pltpu.SemaphoreType.DMA,
        pltpu.SemaphoreType.DMA,
        pltpu.SemaphoreType.DMA,
    ],
)


def _comb_body(acc_ref, den_ref, out_ref):
    a = acc_ref[0] + acc_ref[1]
    dsum = den_ref[0] + den_ref[1]
    out_ref[...] = a / (dsum[:, None] + 1e-16)


_comb_call = pl.pallas_call(
    _comb_body,
    out_shape=jax.ShapeDtypeStruct((N, D), jnp.float32),
)


@jax.jit
def kernel(x, edge_index, W, a_l, a_r):
    al = a_l.reshape(-1).astype(jnp.float32)
    ar = a_r.reshape(-1).astype(jnp.float32)
    h, hl, hr, h16 = _pre_call(x, W, al, ar)
    row4 = edge_index[0].reshape(NW, NG, GS, CH)
    col4 = edge_index[1].reshape(NW, NG, GS, CH)
    acc, den = _sc_call(h, row4, col4, hl, hr, h16)
    return _comb_call(acc.reshape(NC, N, D), den)
